# baseline (speedup 1.0000x reference)
import functools

import jax
import jax.numpy as jnp
from jax import lax
from jax.experimental import pallas as pl
from jax.experimental.pallas import tpu as pltpu
from jax.experimental.pallas import tpu_sc as plsc

_LANES = 16
_CHUNK = 64


@functools.partial(jax.jit, static_argnums=(2, 3, 4))
def _positional_gather(input_len, pos_table, B, T, D):
    NC = 2
    NS = 16
    NW = NC * NS
    rows_per_w = (B * T) // NW
    n_rounds = rows_per_w // _CHUNK
    w_per_b = NW // B

    mesh = plsc.VectorSubcoreMesh(core_axis_name="c", subcore_axis_name="s")

    @functools.partial(
        pl.kernel,
        mesh=mesh,
        out_type=jax.ShapeDtypeStruct((B * T, D), jnp.float32),
        scratch_types=[
            pltpu.VMEM((_LANES,), jnp.int32),
            pltpu.VMEM((_CHUNK,), jnp.int32),
            pltpu.VMEM((_CHUNK, D), jnp.float32),
            pltpu.SemaphoreType.DMA,
        ],
    )
    def _k(len_hbm, table_hbm, out_hbm, lens_v, idx_v, rows_v, sem):
        c = lax.axis_index("c")
        s = lax.axis_index("s")
        wid = s * NC + c
        b = wid // w_per_b
        base_t = (wid % w_per_b) * rows_per_w

        pltpu.sync_copy(len_hbm, lens_v.at[pl.ds(0, B)])
        lens16 = lens_v[...]
        len_b = lens16[0]
        for bb in range(1, B):
            len_b = jnp.where(b == bb, lens16[bb], len_b)

        def round_body(r, carry):
            t0 = base_t + r * _CHUNK
            for g in range(_CHUNK // _LANES):
                t_vec = lax.iota(jnp.int32, 16) + (t0 + g * _LANES)
                idx_v[pl.ds(g * _LANES, _LANES)] = jnp.where(
                    t_vec < len_b, t_vec + 1, 0)
            pltpu.async_copy(table_hbm.at[idx_v], rows_v, sem).wait()
            pltpu.sync_copy(rows_v, out_hbm.at[pl.ds(b * T + t0, _CHUNK)])
            return carry

        lax.fori_loop(0, n_rounds, round_body, 0)

    return _k(input_len, pos_table)


def kernel(input_len, max_len, pos_table):
    del max_len
    V, D = pos_table.shape
    T = V - 1
    B = input_len.shape[0]
    out = _positional_gather(input_len, pos_table, B, T, D)
    return out.reshape(B, T, D)


# speedup vs baseline: 2.3209x; 2.3209x over previous
import functools

import jax
import jax.numpy as jnp
from jax import lax
from jax.experimental import pallas as pl
from jax.experimental.pallas import tpu as pltpu
from jax.experimental.pallas import tpu_sc as plsc

_LANES = 16
_CHUNK = 64


@functools.partial(jax.jit, static_argnums=(2, 3, 4))
def _positional_gather(input_len, pos_table, B, T, D):
    NC = 2
    NS = 16
    NW = NC * NS
    rows_per_w = (B * T) // NW
    n_rounds = rows_per_w // _CHUNK
    w_per_b = NW // B
    C = _CHUNK

    mesh = plsc.VectorSubcoreMesh(core_axis_name="c", subcore_axis_name="s")

    @functools.partial(
        pl.kernel,
        mesh=mesh,
        out_type=jax.ShapeDtypeStruct((B * T, D), jnp.float32),
        scratch_types=[
            pltpu.VMEM((_LANES,), jnp.int32),
            pltpu.VMEM((C + 8, D), jnp.float32),
            pltpu.SemaphoreType.DMA,
            pltpu.SemaphoreType.DMA,
        ],
    )
    def _k(len_hbm, table_hbm, out_hbm, lens_v, buf, semA, semB):
        c = lax.axis_index("c")
        s = lax.axis_index("s")
        wid = s * NC + c
        b = wid // w_per_b
        base_t = (wid % w_per_b) * rows_per_w

        pltpu.sync_copy(len_hbm, lens_v.at[pl.ds(0, B)])
        lens16 = lens_v[...]
        len_b = lens16[0]
        for bb in range(1, B):
            len_b = jnp.where(b == bb, lens16[bb], len_b)

        zero16 = jnp.zeros((_LANES,), jnp.float32)

        def chunk_body(r, zeroed):
            t0 = base_t + r * C
            m = jnp.clip(len_b - t0, 0, C)
            o0 = b * T + t0

            @pl.when(m > 0)
            def _valid():
                cA = pltpu.make_async_copy(
                    table_hbm.at[pl.ds(t0, C)], buf.at[pl.ds(0, C)], semA)
                cB = pltpu.make_async_copy(
                    table_hbm.at[pl.ds(t0 + C, 1)], buf.at[pl.ds(C, 1)], semB)
                cA.start()
                cB.start()
                cA.wait()
                cB.wait()

                def row_body(rp, carry):
                    valid = rp < m
                    for g in range(D // _LANES):
                        v = buf[rp + 1, pl.ds(g * _LANES, _LANES)]
                        buf[rp, pl.ds(g * _LANES, _LANES)] = jnp.where(
                            valid, v, zero16)
                    return carry

                lax.fori_loop(0, C, row_body, 0)
                pltpu.sync_copy(buf.at[pl.ds(0, C)], out_hbm.at[pl.ds(o0, C)])

            @pl.when(m == 0)
            def _pad():
                @pl.when(zeroed == 0)
                def _z():
                    def zrow(rp, carry):
                        for g in range(D // _LANES):
                            buf[rp, pl.ds(g * _LANES, _LANES)] = zero16
                        return carry

                    lax.fori_loop(0, C, zrow, 0)

                pltpu.sync_copy(buf.at[pl.ds(0, C)], out_hbm.at[pl.ds(o0, C)])

            return jnp.where(m == 0, 1, zeroed)

        lax.fori_loop(0, n_rounds, chunk_body, 0)

    return _k(input_len, pos_table)


def kernel(input_len, max_len, pos_table):
    del max_len
    V, D = pos_table.shape
    T = V - 1
    B = input_len.shape[0]
    out = _positional_gather(input_len, pos_table, B, T, D)
    return out.reshape(B, T, D)
